# baseline (speedup 1.0000x reference)
import functools

import jax
import jax.numpy as jnp
from jax import lax
from jax.experimental import pallas as pl
from jax.experimental.pallas import tpu as pltpu
from jax.experimental.pallas import tpu_sc as plsc

B, S, D = 4, 4096, 2048
N_SENTS = 128
R = B * N_SENTS
NC, NS, L = 2, 16, 16
NW = NC * NS
RPW = R // NW
G = RPW // 2
CHUNKS = D // L
UNROLL = 16

_mesh = plsc.VectorSubcoreMesh(core_axis_name="c", subcore_axis_name="s")


@functools.partial(
    pl.kernel,
    mesh=_mesh,
    out_type=jax.ShapeDtypeStruct((R, D), jnp.float32),
    scratch_types=[
        pltpu.VMEM((RPW,), jnp.int32),
        pltpu.VMEM((RPW, L), jnp.float32),
        pltpu.VMEM((RPW, D), jnp.float32),
        pltpu.SemaphoreType.DMA,
        pltpu.SemaphoreType.DMA,
        pltpu.SemaphoreType.DMA,
        pltpu.SemaphoreType.DMA,
        pltpu.SemaphoreType.DMA,
    ],
)
def _gather_pool(wv_hbm, ids_hbm, maskrep_hbm, out_hbm,
                 idx_v, maskr_v, rows_v, sem_i, sem_m, sem_g0, sem_g1, sem_w):
    wid = lax.axis_index("s") * NC + lax.axis_index("c")
    base = wid * RPW
    cp_i = pltpu.async_copy(ids_hbm.at[pl.ds(base, RPW)], idx_v, sem_i)
    cp_m = pltpu.async_copy(maskrep_hbm.at[pl.ds(base, RPW)], maskr_v, sem_m)
    cp_i.wait()
    b = wid // (N_SENTS // RPW)
    idx_v[...] = idx_v[...] + b * S
    cp_g0 = pltpu.async_copy(
        wv_hbm.at[idx_v.at[pl.ds(0, G)]], rows_v.at[pl.ds(0, G)], sem_g0)
    cp_g1 = pltpu.async_copy(
        wv_hbm.at[idx_v.at[pl.ds(G, G)]], rows_v.at[pl.ds(G, G)], sem_g1)
    cp_m.wait()

    def mul_rows(lo, hi):
        for j in range(lo, hi):
            mrow = maskr_v[j, :]

            def body(c, _, j=j, mrow=mrow):
                off = c * (UNROLL * L)
                for u in range(UNROLL):
                    sl = pl.ds(off + u * L, L)
                    rows_v[j, sl] = rows_v[j, sl] * mrow
                return 0

            lax.fori_loop(0, CHUNKS // UNROLL, body, 0)

    cp_g0.wait()
    mul_rows(0, G)
    cp_w0 = pltpu.async_copy(
        rows_v.at[pl.ds(0, G)], out_hbm.at[pl.ds(base, G)], sem_w)
    cp_g1.wait()
    mul_rows(G, RPW)
    cp_w1 = pltpu.async_copy(
        rows_v.at[pl.ds(G, G)], out_hbm.at[pl.ds(base + G, G)], sem_g1)
    cp_w0.wait()
    cp_w1.wait()


def kernel(word_vectors, sent_rep_token_ids, sent_rep_mask):
    table = word_vectors.reshape(B * S, D)
    ids = sent_rep_token_ids.reshape(R)
    maskrep = jnp.broadcast_to(
        sent_rep_mask.astype(jnp.float32).reshape(R, 1), (R, L))
    out = _gather_pool(table, ids, maskrep)
    return out.reshape(B, N_SENTS, D), sent_rep_mask


# speedup vs baseline: 1.0274x; 1.0274x over previous
import functools

import jax
import jax.numpy as jnp
from jax import lax
from jax.experimental import pallas as pl
from jax.experimental.pallas import tpu as pltpu
from jax.experimental.pallas import tpu_sc as plsc

B, S, D = 4, 4096, 2048
N_SENTS = 128
R = B * N_SENTS
NC, NS, L = 2, 16, 16
NW = NC * NS
RPW = R // NW
G = RPW // 2
CHUNKS = D // L
UNROLL = 8

_mesh = plsc.VectorSubcoreMesh(core_axis_name="c", subcore_axis_name="s")


@functools.partial(
    pl.kernel,
    mesh=_mesh,
    out_type=jax.ShapeDtypeStruct((R, D), jnp.float32),
    scratch_types=[
        pltpu.VMEM((RPW,), jnp.int32),
        pltpu.VMEM((RPW, L), jnp.float32),
        pltpu.VMEM((RPW, D), jnp.float32),
        pltpu.SemaphoreType.DMA,
        pltpu.SemaphoreType.DMA,
        pltpu.SemaphoreType.DMA,
        pltpu.SemaphoreType.DMA,
        pltpu.SemaphoreType.DMA,
    ],
)
def _gather_pool(wv_hbm, ids_hbm, maskrep_hbm, out_hbm,
                 idx_v, maskr_v, rows_v, sem_i, sem_m, sem_g0, sem_g1, sem_w):
    wid = lax.axis_index("s") * NC + lax.axis_index("c")
    base = wid * RPW
    cp_i = pltpu.async_copy(ids_hbm.at[pl.ds(base, RPW)], idx_v, sem_i)
    cp_m = pltpu.async_copy(maskrep_hbm.at[pl.ds(base, RPW)], maskr_v, sem_m)
    cp_i.wait()
    b = wid // (N_SENTS // RPW)
    idx_v[...] = idx_v[...] + b * S
    cp_g0 = pltpu.async_copy(
        wv_hbm.at[idx_v.at[pl.ds(0, G)]], rows_v.at[pl.ds(0, G)], sem_g0)
    cp_g1 = pltpu.async_copy(
        wv_hbm.at[idx_v.at[pl.ds(G, G)]], rows_v.at[pl.ds(G, G)], sem_g1)
    cp_m.wait()

    def mul_rows(lo, hi):
        for j in range(lo, hi):
            mrow = maskr_v[j, :]

            def body(c, _, j=j, mrow=mrow):
                off = c * (UNROLL * L)
                for u in range(UNROLL):
                    sl = pl.ds(off + u * L, L)
                    rows_v[j, sl] = rows_v[j, sl] * mrow
                return 0

            lax.fori_loop(0, CHUNKS // UNROLL, body, 0)

    cp_g0.wait()
    mul_rows(0, G)
    cp_w0 = pltpu.async_copy(
        rows_v.at[pl.ds(0, G)], out_hbm.at[pl.ds(base, G)], sem_w)
    cp_g1.wait()
    mul_rows(G, RPW)
    cp_w1 = pltpu.async_copy(
        rows_v.at[pl.ds(G, G)], out_hbm.at[pl.ds(base + G, G)], sem_g1)
    cp_w0.wait()
    cp_w1.wait()


def kernel(word_vectors, sent_rep_token_ids, sent_rep_mask):
    table = word_vectors.reshape(B * S, D)
    ids = sent_rep_token_ids.reshape(R)
    maskrep = jnp.broadcast_to(
        sent_rep_mask.astype(jnp.float32).reshape(R, 1), (R, L))
    out = _gather_pool(table, ids, maskrep)
    return out.reshape(B, N_SENTS, D), sent_rep_mask
